# baseline (speedup 1.0000x reference)
import functools

import jax
import jax.numpy as jnp
from jax import lax
from jax.experimental import pallas as pl
from jax.experimental.pallas import tpu as pltpu
from jax.experimental.pallas import tpu_sc as plsc

BATCH = 16384
EMBED = 64
PAIR = 2 * EMBED
NUM_CORES = 2
NUM_SUBCORES = 16
NUM_WORKERS = NUM_CORES * NUM_SUBCORES
B_PER_W = BATCH // NUM_WORKERS
CHUNK = 128
NCHUNK = B_PER_W // CHUNK
GROUPS = CHUNK // 16

_mesh = plsc.VectorSubcoreMesh(core_axis_name="c", subcore_axis_name="s")


@functools.partial(
    pl.kernel,
    out_type=jax.ShapeDtypeStruct((BATCH,), jnp.float32),
    mesh=_mesh,
    compiler_params=pltpu.CompilerParams(needs_layout_passes=False),
    scratch_types=[
        pltpu.VMEM((NCHUNK, CHUNK), jnp.int32),
        pltpu.VMEM((B_PER_W,), jnp.int32),
        [pltpu.VMEM((CHUNK, PAIR), jnp.float32) for _ in range(NCHUNK)],
        pltpu.VMEM((B_PER_W,), jnp.float32),
        pltpu.VMEM((EMBED,), jnp.float32),
        pltpu.VMEM((16,), jnp.float32),
        pltpu.SemaphoreType.DMA,
    ],
)
def _sc_kernel(ids_hbm, table2_hbm, w_hbm, b_hbm, out_hbm,
               idx_v, colb_v, rows_v, out_v, w_v, b_v, sem):
    wid = lax.axis_index("s") * NUM_CORES + lax.axis_index("c")
    base = pl.multiple_of(wid * B_PER_W, B_PER_W)

    pltpu.sync_copy(w_hbm.at[0], w_v)
    pltpu.sync_copy(b_hbm, b_v.at[pl.ds(0, 1)])
    for c in range(NCHUNK):
        pltpu.sync_copy(ids_hbm.at[pl.ds(base + c * CHUNK, CHUNK)],
                        idx_v.at[c])

    copies = []
    for c in range(NCHUNK):
        def cbody(j, _, c=c):
            joff = pl.multiple_of(j * 16, 16)
            ids16 = idx_v[c, pl.ds(joff, 16)]
            colb_v[pl.ds(c * CHUNK + joff, 16)] = (ids16 & 1) * EMBED
            idx_v[c, pl.ds(joff, 16)] = ids16 >> 1
            return 0
        lax.fori_loop(0, GROUPS, cbody, 0)
        copies.append(
            pltpu.async_copy(table2_hbm.at[idx_v.at[c]], rows_v[c], sem))
    for cp in copies:
        cp.wait()

    b_s = b_v[pl.ds(0, 16)][0]
    w_vecs = [w_v[pl.ds(k * 16, 16)] for k in range(EMBED // 16)]
    w_s = [w_vecs[d // 16][d % 16] for d in range(EMBED)]
    lane = lax.iota(jnp.int32, 16)

    for c in range(NCHUNK):
        rows_c = rows_v[c]

        def body(g, _, rows_c=rows_c, c=c):
            goff = pl.multiple_of(g * 16, 16)
            row_ids = g * 16 + lane
            colb = colb_v[pl.ds(c * CHUNK + goff, 16)]
            acc = jnp.full((16,), b_s, jnp.float32)
            for d in range(EMBED):
                v = plsc.load_gather(rows_c, [row_ids, colb + d])
                acc = acc + v * w_s[d]
            out_v[pl.ds(c * CHUNK + goff, 16)] = acc
            return 0

        lax.fori_loop(0, GROUPS, body, 0)

    pltpu.sync_copy(out_v, out_hbm.at[pl.ds(base, B_PER_W)])


def kernel(dataset_ids, table, W, b):
    table2 = table.reshape(table.shape[0] // 2, PAIR)
    return _sc_kernel(dataset_ids.astype(jnp.int32), table2, W, b)


# speedup vs baseline: 3.1016x; 3.1016x over previous
import functools

import jax
import jax.numpy as jnp
from jax import lax
from jax.experimental import pallas as pl
from jax.experimental.pallas import tpu as pltpu
from jax.experimental.pallas import tpu_sc as plsc

BATCH = 16384
EMBED = 64
NUM_DATASETS = 1000000
BC = 4096
GRID = (NUM_DATASETS + BC - 1) // BC

NUM_CORES = 2
NUM_SUBCORES = 16
NUM_WORKERS = NUM_CORES * NUM_SUBCORES
B_PER_W = BATCH // NUM_WORKERS
CHUNK = 128
NCHUNK = B_PER_W // CHUNK


def _mv_body(w_ref, b_ref, t_ref, o_ref):
    x = t_ref[...]
    w = w_ref[...].reshape(EMBED, 1)
    o_ref[...] = (x * w).sum(axis=0) + b_ref[0]


_matvec = pl.pallas_call(
    _mv_body,
    grid=(GRID,),
    in_specs=[
        pl.BlockSpec((1, EMBED), lambda i: (0, 0)),
        pl.BlockSpec(memory_space=pltpu.SMEM),
        pl.BlockSpec((EMBED, BC), lambda i: (0, i)),
    ],
    out_specs=pl.BlockSpec((BC,), lambda i: (i,)),
    out_shape=jax.ShapeDtypeStruct((NUM_DATASETS,), jnp.float32),
)

_mesh = plsc.VectorSubcoreMesh(core_axis_name="c", subcore_axis_name="s")


@functools.partial(
    pl.kernel,
    out_type=jax.ShapeDtypeStruct((BATCH,), jnp.float32),
    mesh=_mesh,
    compiler_params=pltpu.CompilerParams(
        needs_layout_passes=False, use_tc_tiling_on_sc=False),
    scratch_types=[
        pltpu.VMEM((NCHUNK, CHUNK), jnp.int32),
        pltpu.VMEM((B_PER_W,), jnp.float32),
        pltpu.SemaphoreType.DMA,
    ],
)
def _sc_gather(ids_hbm, s_hbm, out_hbm, idx_v, out_v, sem):
    wid = lax.axis_index("s") * NUM_CORES + lax.axis_index("c")
    base = pl.multiple_of(wid * B_PER_W, B_PER_W)

    for c in range(NCHUNK):
        pltpu.sync_copy(ids_hbm.at[pl.ds(base + c * CHUNK, CHUNK)],
                        idx_v.at[c])
    copies = []
    for c in range(NCHUNK):
        copies.append(
            pltpu.async_copy(s_hbm.at[idx_v.at[c]],
                             out_v.at[pl.ds(c * CHUNK, CHUNK)], sem))
    for cp in copies:
        cp.wait()
    pltpu.sync_copy(out_v, out_hbm.at[pl.ds(base, B_PER_W)])


def kernel(dataset_ids, table, W, b):
    s = _matvec(W, b, table.T)
    return _sc_gather(dataset_ids.astype(jnp.int32), s)


# speedup vs baseline: 5.5904x; 1.8024x over previous
import functools

import jax
import jax.numpy as jnp
from jax import lax
from jax.experimental import pallas as pl
from jax.experimental.pallas import tpu as pltpu
from jax.experimental.pallas import tpu_sc as plsc

BATCH = 16384
EMBED = 64
NUM_DATASETS = 1000000
BC = 16384
GRID = (NUM_DATASETS + BC - 1) // BC

NUM_CORES = 2
NUM_SUBCORES = 16
NUM_WORKERS = NUM_CORES * NUM_SUBCORES
B_PER_W = BATCH // NUM_WORKERS
CHUNK = 128
NCHUNK = B_PER_W // CHUNK


def _mv_body(w_ref, b_ref, t_ref, o_ref):
    x = t_ref[...]
    w = w_ref[...].reshape(EMBED, 1)
    o_ref[...] = (x * w).sum(axis=0) + b_ref[0]


_matvec = pl.pallas_call(
    _mv_body,
    grid=(GRID,),
    in_specs=[
        pl.BlockSpec((1, EMBED), lambda i: (0, 0)),
        pl.BlockSpec(memory_space=pltpu.SMEM),
        pl.BlockSpec((EMBED, BC), lambda i: (0, i)),
    ],
    out_specs=pl.BlockSpec((BC,), lambda i: (i,)),
    out_shape=jax.ShapeDtypeStruct((NUM_DATASETS,), jnp.float32),
)

_mesh = plsc.VectorSubcoreMesh(core_axis_name="c", subcore_axis_name="s")


@functools.partial(
    pl.kernel,
    out_type=jax.ShapeDtypeStruct((BATCH,), jnp.float32),
    mesh=_mesh,
    compiler_params=pltpu.CompilerParams(
        needs_layout_passes=False, use_tc_tiling_on_sc=False),
    scratch_types=[
        pltpu.VMEM((NCHUNK, CHUNK), jnp.int32),
        pltpu.VMEM((B_PER_W,), jnp.float32),
        pltpu.SemaphoreType.DMA,
    ],
)
def _sc_gather(ids_hbm, s_hbm, out_hbm, idx_v, out_v, sem):
    wid = lax.axis_index("s") * NUM_CORES + lax.axis_index("c")
    base = pl.multiple_of(wid * B_PER_W, B_PER_W)

    for c in range(NCHUNK):
        pltpu.sync_copy(ids_hbm.at[pl.ds(base + c * CHUNK, CHUNK)],
                        idx_v.at[c])
    copies = []
    for c in range(NCHUNK):
        copies.append(
            pltpu.async_copy(s_hbm.at[idx_v.at[c]],
                             out_v.at[pl.ds(c * CHUNK, CHUNK)], sem))
    for cp in copies:
        cp.wait()
    pltpu.sync_copy(out_v, out_hbm.at[pl.ds(base, B_PER_W)])


def kernel(dataset_ids, table, W, b):
    s = _matvec(W, b, table.T)
    return _sc_gather(dataset_ids.astype(jnp.int32), s)


# speedup vs baseline: 6.4226x; 1.1489x over previous
import functools

import jax
import jax.numpy as jnp
from jax import lax
from jax.experimental import pallas as pl
from jax.experimental.pallas import tpu as pltpu
from jax.experimental.pallas import tpu_sc as plsc

BATCH = 16384
EMBED = 64
NUM_DATASETS = 1000000
BC = 32768
GRID = (NUM_DATASETS + BC - 1) // BC

NUM_CORES = 2
NUM_SUBCORES = 16
NUM_WORKERS = NUM_CORES * NUM_SUBCORES
B_PER_W = BATCH // NUM_WORKERS
CHUNK = 128
NCHUNK = B_PER_W // CHUNK


def _mv_body(w_ref, b_ref, t_ref, o_ref):
    x = t_ref[...]
    w = w_ref[...].reshape(EMBED, 1)
    o_ref[...] = (x * w).sum(axis=0) + b_ref[0]


_matvec = pl.pallas_call(
    _mv_body,
    grid=(GRID,),
    in_specs=[
        pl.BlockSpec((1, EMBED), lambda i: (0, 0)),
        pl.BlockSpec(memory_space=pltpu.SMEM),
        pl.BlockSpec((EMBED, BC), lambda i: (0, i)),
    ],
    out_specs=pl.BlockSpec((BC,), lambda i: (i,)),
    out_shape=jax.ShapeDtypeStruct((NUM_DATASETS,), jnp.float32),
)

_mesh = plsc.VectorSubcoreMesh(core_axis_name="c", subcore_axis_name="s")


@functools.partial(
    pl.kernel,
    out_type=jax.ShapeDtypeStruct((BATCH,), jnp.float32),
    mesh=_mesh,
    compiler_params=pltpu.CompilerParams(
        needs_layout_passes=False, use_tc_tiling_on_sc=False),
    scratch_types=[
        pltpu.VMEM((NCHUNK, CHUNK), jnp.int32),
        pltpu.VMEM((B_PER_W,), jnp.float32),
        pltpu.SemaphoreType.DMA,
    ],
)
def _sc_gather(ids_hbm, s_hbm, out_hbm, idx_v, out_v, sem):
    wid = lax.axis_index("s") * NUM_CORES + lax.axis_index("c")
    base = pl.multiple_of(wid * B_PER_W, B_PER_W)

    for c in range(NCHUNK):
        pltpu.sync_copy(ids_hbm.at[pl.ds(base + c * CHUNK, CHUNK)],
                        idx_v.at[c])
    copies = []
    for c in range(NCHUNK):
        copies.append(
            pltpu.async_copy(s_hbm.at[idx_v.at[c]],
                             out_v.at[pl.ds(c * CHUNK, CHUNK)], sem))
    for cp in copies:
        cp.wait()
    pltpu.sync_copy(out_v, out_hbm.at[pl.ds(base, B_PER_W)])


def kernel(dataset_ids, table, W, b):
    s = _matvec(W, b, table.T)
    return _sc_gather(dataset_ids.astype(jnp.int32), s)
